# baseline (speedup 1.0000x reference)
import functools

import jax
import jax.numpy as jnp
from jax import lax
from jax.experimental import pallas as pl
from jax.experimental.pallas import tpu as pltpu
from jax.experimental.pallas import tpu_sc as plsc

BATCH = 4096
SEQ_LEN = 200
VOCAB = 6
DIM = 64

N_ROWS = BATCH * SEQ_LEN
NC, NS = 2, 16
NW = NC * NS
ROWS_PER_W = N_ROWS // NW
CHUNK_ROWS = 512
N_CHUNKS = ROWS_PER_W // CHUNK_ROWS
L = 16


def _mesh():
    return plsc.VectorSubcoreMesh(
        core_axis_name="c", subcore_axis_name="s",
        num_cores=NC, num_subcores=NS)


@functools.partial(
    pl.kernel,
    out_type=jax.ShapeDtypeStruct((N_ROWS * DIM,), jnp.float32),
    mesh=_mesh(),
    compiler_params=pltpu.CompilerParams(use_tc_tiling_on_sc=True),
    scratch_types=[
        pltpu.VMEM((VOCAB * DIM,), jnp.float32),
        pltpu.VMEM((ROWS_PER_W,), jnp.int32),
        pltpu.VMEM((2, CHUNK_ROWS * DIM), jnp.float32),
        pltpu.SemaphoreType.DMA,
        pltpu.SemaphoreType.DMA,
    ],
)
def _expand_kernel(table_hbm, idx_hbm, out_hbm, table_v, idx_v, rows_v,
                   sem_o0, sem_o1):
    wid = lax.axis_index("s") * NC + lax.axis_index("c")
    r_base = wid * ROWS_PER_W
    sem_o = (sem_o0, sem_o1)

    pltpu.sync_copy(table_hbm, table_v)
    pltpu.sync_copy(idx_hbm.at[pl.ds(r_base, ROWS_PER_W)], idx_v)

    @pl.loop(0, N_CHUNKS, step=2)
    def chunk_pair(t):
        for b in range(2):
            k = t + b
            r0 = r_base + k * CHUNK_ROWS

            @pl.when(t >= 2)
            def _wait_out():
                pltpu.make_async_copy(
                    rows_v.at[b],
                    out_hbm.at[pl.ds(r0 * DIM, CHUNK_ROWS * DIM)],
                    sem_o[b]).wait()

            @plsc.parallel_loop(0, CHUNK_ROWS // L, unroll=2)
            def expand(g):
                offs = idx_v[pl.ds(k * CHUNK_ROWS + g * L, L)] * DIM
                d0 = g * (L * DIM)
                for kk in range(L):
                    o = offs[kk]
                    d = d0 + kk * DIM
                    for c in range(DIM // L):
                        rows_v[b, pl.ds(d + c * L, L)] = (
                            table_v[pl.ds(o + c * L, L)])

            pltpu.async_copy(
                rows_v.at[b],
                out_hbm.at[pl.ds(r0 * DIM, CHUNK_ROWS * DIM)], sem_o[b])

    for b in range(2):
        pltpu.make_async_copy(
            rows_v.at[b],
            out_hbm.at[pl.ds(r_base * DIM, CHUNK_ROWS * DIM)],
            sem_o[b]).wait()


def kernel(indices, table):
    idx = indices.reshape(N_ROWS).astype(jnp.int32)
    flat = _expand_kernel(table.reshape(VOCAB * DIM), idx)
    return flat.reshape(N_ROWS, DIM)


# speedup vs baseline: 2.4684x; 2.4684x over previous
import functools

import jax
import jax.numpy as jnp
from jax import lax
from jax.experimental import pallas as pl
from jax.experimental.pallas import tpu as pltpu
from jax.experimental.pallas import tpu_sc as plsc

BATCH = 4096
SEQ_LEN = 200
VOCAB = 6
DIM = 64

N_ROWS = BATCH * SEQ_LEN
NC, NS = 2, 16
NW = NC * NS
ROWS_PER_W = N_ROWS // NW
CHUNK_ROWS = 512
N_CHUNKS = ROWS_PER_W // CHUNK_ROWS
L = 16
REP = 17


def _mesh():
    return plsc.VectorSubcoreMesh(
        core_axis_name="c", subcore_axis_name="s",
        num_cores=NC, num_subcores=NS)


@functools.partial(
    pl.kernel,
    out_type=jax.ShapeDtypeStruct((DIM, N_ROWS), jnp.float32),
    mesh=_mesh(),
    compiler_params=pltpu.CompilerParams(
        use_tc_tiling_on_sc=True, needs_layout_passes=False),
    scratch_types=[
        pltpu.VMEM((VOCAB * DIM,), jnp.float32),
        pltpu.VMEM((DIM * L * REP,), jnp.float32),
        pltpu.VMEM((ROWS_PER_W,), jnp.int32),
        pltpu.VMEM((2, DIM, CHUNK_ROWS), jnp.float32),
        pltpu.SemaphoreType.DMA,
        pltpu.SemaphoreType.DMA,
    ],
)
def _expand_kernel(table_hbm, idx_hbm, out_hbm, table_v, rep_v, idx_v,
                   cols_v, sem_o0, sem_o1):
    wid = lax.axis_index("s") * NC + lax.axis_index("c")
    r_base = wid * ROWS_PER_W
    sem_o = (sem_o0, sem_o1)

    pltpu.sync_copy(table_hbm, table_v)
    pltpu.sync_copy(idx_hbm.at[pl.ds(r_base, ROWS_PER_W)], idx_v)

    iota = lax.iota(jnp.int32, L)
    c_stride = iota * (L * REP)
    for v in range(VOCAB):
        for cb in range(DIM // L):
            tv = table_v[pl.ds(v * DIM + cb * L, L)]
            for l in range(L):
                addr = c_stride + (cb * L * (L * REP) + l * REP + v)
                plsc.store_scatter(rep_v, [addr], tv)

    lane_rep = iota * REP

    @pl.loop(0, N_CHUNKS, step=2)
    def chunk_pair(t):
        for b in range(2):
            k = t + b
            r0 = r_base + k * CHUNK_ROWS

            @pl.when(t >= 2)
            def _wait_out():
                pltpu.make_async_copy(
                    cols_v.at[b],
                    out_hbm.at[:, pl.ds(r0, CHUNK_ROWS)],
                    sem_o[b]).wait()

            @plsc.parallel_loop(0, CHUNK_ROWS // L, unroll=2)
            def expand(g):
                base = idx_v[pl.ds(k * CHUNK_ROWS + g * L, L)] + lane_rep
                for c in range(DIM):
                    val = plsc.load_gather(rep_v, [base + c * (L * REP)])
                    cols_v[b, c, pl.ds(g * L, L)] = val

            pltpu.async_copy(
                cols_v.at[b],
                out_hbm.at[:, pl.ds(r0, CHUNK_ROWS)], sem_o[b])

    for b in range(2):
        pltpu.make_async_copy(
            cols_v.at[b],
            out_hbm.at[:, pl.ds(r_base, CHUNK_ROWS)],
            sem_o[b]).wait()


def kernel(indices, table):
    idx = indices.reshape(N_ROWS).astype(jnp.int32)
    out_t = _expand_kernel(table.reshape(VOCAB * DIM), idx)
    return out_t.T
